# Initial kernel scaffold; baseline (speedup 1.0000x reference)
import jax
import jax.numpy as jnp
from jax.experimental import pallas as pl


def kernel(tokens, table):
    raise NotImplementedError("write your pallas kernel here")


# speedup vs baseline: 4.5679x; 4.5679x over previous
import functools
import math

import jax
import jax.numpy as jnp
from jax import lax
from jax.experimental import pallas as pl
from jax.experimental.pallas import tpu as pltpu
from jax.experimental.pallas import tpu_sc as plsc

_EMB = 32
_SCALE = math.sqrt(_EMB)
_LANES = 16


def _make_sc_kernel(B: int, V: int, chunk: int):
    info = plsc.get_sparse_core_info()
    NC, NS = info.num_cores, info.num_subcores
    NW = NC * NS
    assert B % NW == 0
    b_per_w = B // NW
    assert b_per_w % chunk == 0
    n_chunks = b_per_w // chunk
    vregs_per_row = _EMB // _LANES

    mesh = plsc.VectorSubcoreMesh(core_axis_name="c", subcore_axis_name="s")

    @functools.partial(
        pl.kernel,
        out_type=jax.ShapeDtypeStruct((B, _EMB), jnp.float32),
        mesh=mesh,
        scratch_types=[
            pltpu.VMEM((chunk,), jnp.int32),
            pltpu.VMEM((chunk, _EMB), jnp.float32),
            pltpu.SemaphoreType.DMA,
        ],
        compiler_params=pltpu.CompilerParams(use_tc_tiling_on_sc=False),
    )
    def k(tokens_hbm, table_hbm, out_hbm, idx_v, rows_v, sem):
        wid = lax.axis_index("s") * NC + lax.axis_index("c")
        base = wid * b_per_w

        def chunk_body(g, carry):
            start = base + g * chunk
            pltpu.sync_copy(tokens_hbm.at[pl.ds(start, chunk)], idx_v)
            pltpu.async_copy(table_hbm.at[idx_v], rows_v, sem).wait()

            def scale_body(i, carry2):
                for j in range(vregs_per_row):
                    sl = pl.ds(j * _LANES, _LANES)
                    rows_v[i, sl] = rows_v[i, sl] * _SCALE
                return carry2

            lax.fori_loop(0, chunk, scale_body, 0, unroll=8)
            pltpu.sync_copy(rows_v, out_hbm.at[pl.ds(start, chunk)])
            return carry

        lax.fori_loop(0, n_chunks, chunk_body, 0)

    return k


@jax.jit
def kernel(tokens, table):
    B = tokens.shape[0] * tokens.shape[1]
    V = table.shape[0]
    flat = jnp.reshape(tokens, (B,))
    k = _make_sc_kernel(B, V, chunk=1024)
    out = k(flat, table)
    return jnp.reshape(out, (*tokens.shape, _EMB))


# speedup vs baseline: 5.0181x; 1.0986x over previous
import functools
import math

import jax
import jax.numpy as jnp
from jax import lax
from jax.experimental import pallas as pl
from jax.experimental.pallas import tpu as pltpu
from jax.experimental.pallas import tpu_sc as plsc

_EMB = 32
_SCALE = math.sqrt(_EMB)
_LANES = 16


def _make_sc_kernel(B: int, chunk: int, S: int, K: int):
    info = plsc.get_sparse_core_info()
    NC, NS = info.num_cores, info.num_subcores
    NW = NC * NS
    assert B % NW == 0
    b_per_w = B // NW
    assert b_per_w % chunk == 0
    n_chunks = b_per_w // chunk
    n_visits = n_chunks + K
    rounds = (n_visits + S - 1) // S
    vregs_per_row = _EMB // _LANES

    mesh = plsc.VectorSubcoreMesh(core_axis_name="c", subcore_axis_name="s")

    @functools.partial(
        pl.kernel,
        out_type=jax.ShapeDtypeStruct((B, _EMB), jnp.float32),
        mesh=mesh,
        scratch_types=[
            [pltpu.VMEM((chunk,), jnp.int32) for _ in range(S)],
            [pltpu.VMEM((chunk, _EMB), jnp.float32) for _ in range(S)],
            [pltpu.SemaphoreType.DMA for _ in range(S)],
            [pltpu.SemaphoreType.DMA for _ in range(S)],
            [pltpu.SemaphoreType.DMA for _ in range(S)],
        ],
        compiler_params=pltpu.CompilerParams(use_tc_tiling_on_sc=False),
    )
    def k(tokens_hbm, table_hbm, out_hbm, idx_v, rows_v, isem, gsem, ssem):
        wid = lax.axis_index("s") * NC + lax.axis_index("c")
        base = wid * b_per_w

        def idx_load(c, slot):
            return pltpu.async_copy(
                tokens_hbm.at[pl.ds(base + c * chunk, chunk)], idx_v[slot],
                isem[slot])

        for b in range(S):
            idx_load(b, b)

        def round_body(r, carry):
            for b in range(S):
                g = r * S + b

                @pl.when(g < n_chunks)
                def _fetch():
                    pltpu.make_async_copy(
                        tokens_hbm.at[pl.ds(base, chunk)], idx_v[b],
                        isem[b]).wait()

                    @pl.when(g >= S)
                    def _drain_prev_scatter():
                        pltpu.make_async_copy(
                            rows_v[b], out_hbm.at[pl.ds(base, chunk)],
                            ssem[b]).wait()

                    pltpu.async_copy(
                        table_hbm.at[idx_v[b]], rows_v[b], gsem[b])

                p = g - K
                bp = (b - K) % S

                @pl.when((g >= K) & (p < n_chunks))
                def _process():
                    pltpu.make_async_copy(
                        table_hbm.at[idx_v[bp]], rows_v[bp], gsem[bp]).wait()

                    @pl.when(p + S < n_chunks)
                    def _prefetch_idx():
                        idx_load(p + S, bp)

                    def scale_body(i, c2):
                        for j in range(vregs_per_row):
                            sl = pl.ds(j * _LANES, _LANES)
                            rows_v[bp][i, sl] = rows_v[bp][i, sl] * _SCALE
                        return c2

                    lax.fori_loop(0, chunk, scale_body, 0, unroll=8)
                    pltpu.async_copy(
                        rows_v[bp], out_hbm.at[pl.ds(base + p * chunk, chunk)],
                        ssem[bp])

            return carry

        lax.fori_loop(0, rounds, round_body, 0)

        for b in range(S):
            c = n_chunks - S + b
            pltpu.make_async_copy(
                rows_v[b], out_hbm.at[pl.ds(base, chunk)], ssem[b]).wait()

    return k


@jax.jit
def kernel(tokens, table):
    B = tokens.shape[0] * tokens.shape[1]
    flat = jnp.reshape(tokens, (B,))
    k = _make_sc_kernel(B, chunk=512, S=5, K=2)
    out = k(flat, table)
    return jnp.reshape(out, (*tokens.shape, _EMB))
